# Initial kernel scaffold; baseline (speedup 1.0000x reference)
import jax
import jax.numpy as jnp
from jax.experimental import pallas as pl


def kernel(x, Wp1, bp1, Wp2, bp2, Wm1, bm1, Wm2, bm2):
    raise NotImplementedError("write your pallas kernel here")


# speedup vs baseline: 20.3757x; 20.3757x over previous
import functools

import jax
import jax.numpy as jnp
from jax import lax
from jax.experimental import pallas as pl
from jax.experimental.pallas import tpu as pltpu

_TOPK = 256
_TV = 2048
_RB = 32
_RBC = 256


def _mono(bits):
    return jnp.where(bits >= 0, bits, bits ^ jnp.int32(0x7FFFFFFF))


def _h_body(x_ref, w1p_ref, b1p_ref, w1m_ref, b1m_ref, hp_ref, hm_ref):
    x = x_ref[...]
    for w1, b1, h in ((w1p_ref, b1p_ref, hp_ref), (w1m_ref, b1m_ref, hm_ref)):
        pre = jnp.dot(x, w1[...], preferred_element_type=jnp.float32) + b1[...]
        h[...] = 0.5 * pre * (1.0 + lax.erf(pre * 0.7071067811865476))


def _z_body(hp_ref, w2p_ref, b2p_ref, hm_ref, w2m_ref, b2m_ref, up_ref, um_ref):
    for h, w2, b2, u in ((hp_ref, w2p_ref, b2p_ref, up_ref),
                         (hm_ref, w2m_ref, b2m_ref, um_ref)):
        z = jnp.dot(h[...], w2[...], preferred_element_type=jnp.float32) + b2[...]
        u[...] = _mono(lax.bitcast_convert_type(z, jnp.int32))


def _tau_body(u_ref, tau_ref, *, k):
    u = u_ref[...]

    def count_ge(t):
        return jnp.sum((u >= t).astype(jnp.int32), axis=1, keepdims=True)

    rb = u.shape[0]
    t0 = jnp.where(count_ge(jnp.zeros((rb, 1), jnp.int32)) >= k,
                   jnp.int32(0), jnp.int32(-2147483648))

    def step(i, t):
        cand = t | (jnp.int32(1) << (jnp.int32(30) - i))
        return jnp.where(count_ge(cand) >= k, cand, t)

    t = lax.fori_loop(0, 31, step, t0)
    tau_ref[0, :, :] = t


def _out_body(tau_ref, u_ref, o_ref):
    u = u_ref[...]
    keep = u >= tau_ref[...]
    z = lax.bitcast_convert_type(_mono(u), jnp.float32)
    sp = jnp.log1p(jnp.exp(-jnp.abs(z))) + jnp.maximum(z, 0.0)
    o_ref[...] = jnp.where(keep, sp, 0.0)


def kernel(x, Wp1, bp1, Wp2, bp2, Wm1, bm1, Wm2, bm2):
    B, DIN = x.shape
    V = Wp2.shape[1]
    BOT = Wp1.shape[1]
    k = min(_TOPK, V)
    tv = min(_TV, V)
    vp = -(-V // tv) * tv
    rb = min(_RB, B)
    nrb = B // rb
    rbc = min(_RBC, B)

    f32 = jnp.float32
    if vp != V:
        Wp2 = jnp.pad(Wp2, ((0, 0), (0, vp - V)))
        Wm2 = jnp.pad(Wm2, ((0, 0), (0, vp - V)))
        bp2 = jnp.pad(bp2, (0, vp - V), constant_values=-1e30)
        bm2 = jnp.pad(bm2, (0, vp - V), constant_values=-1e30)
    b1p = bp1.reshape(1, BOT)
    b1m = bm1.reshape(1, BOT)
    b2p = bp2.reshape(1, vp)
    b2m = bm2.reshape(1, vp)

    hp, hm = pl.pallas_call(
        _h_body,
        out_shape=[jax.ShapeDtypeStruct((B, BOT), f32)] * 2,
    )(x, Wp1, b1p, Wm1, b1m)

    zspec_w = pl.BlockSpec((DIN, tv), lambda j: (0, j))
    zspec_b = pl.BlockSpec((1, tv), lambda j: (0, j))
    zspec_h = pl.BlockSpec((B, BOT), lambda j: (0, 0))
    zspec_u = pl.BlockSpec((B, tv), lambda j: (0, j))
    up, um = pl.pallas_call(
        _z_body,
        grid=(vp // tv,),
        in_specs=[zspec_h, zspec_w, zspec_b, zspec_h, zspec_w, zspec_b],
        out_specs=[zspec_u, zspec_u],
        out_shape=[jax.ShapeDtypeStruct((B, vp), jnp.int32)] * 2,
    )(hp, Wp2, b2p, hm, Wm2, b2m)

    taus = []
    for u in (up, um):
        tau = pl.pallas_call(
            functools.partial(_tau_body, k=k),
            grid=(nrb,),
            in_specs=[pl.BlockSpec((rb, vp), lambda i: (i, 0))],
            out_specs=pl.BlockSpec((1, rb, 1), lambda i: (i, 0, 0)),
            out_shape=jax.ShapeDtypeStruct((nrb, rb, 1), jnp.int32),
        )(u)
        taus.append(tau.reshape(B, 1))
    tau_p, tau_m = taus

    ospec_t = pl.BlockSpec((rbc, 1), lambda i, j: (i, 0))
    ospec_u = pl.BlockSpec((rbc, tv), lambda i, j: (i, j))
    outs = []
    for tau, u in ((tau_p, up), (tau_m, um)):
        out = pl.pallas_call(
            _out_body,
            grid=(B // rbc, vp // tv),
            in_specs=[ospec_t, ospec_u],
            out_specs=ospec_u,
            out_shape=jax.ShapeDtypeStruct((B, V), f32),
        )(tau, u)
        outs.append(out)
    return tuple(outs)
